# baseline (speedup 1.0000x reference)
import functools

import numpy as np
import jax
import jax.numpy as jnp
from jax import lax
from jax.experimental import pallas as pl
from jax.experimental.pallas import tpu as pltpu
from jax.experimental.pallas import tpu_sc as plsc

_P = 0.1
_LANES = 16
_NUM_WORKERS = 32
_ROWS_PER_CHUNK = 16

_perm_cache = {}


def _rotl32(x, r):
    return ((x << np.uint32(r)) | (x >> np.uint32(32 - r))).astype(np.uint32)


def _threefry2x32(k0, k1, x0, x1):
    rotations = ((13, 15, 26, 6), (17, 29, 16, 24))
    ks = (
        np.uint32(k0),
        np.uint32(k1),
        np.uint32(np.uint32(k0) ^ np.uint32(k1) ^ np.uint32(0x1BD11BDA)),
    )
    x0 = (x0 + ks[0]).astype(np.uint32)
    x1 = (x1 + ks[1]).astype(np.uint32)
    for i in range(5):
        for r in rotations[i % 2]:
            x0 = (x0 + x1).astype(np.uint32)
            x1 = _rotl32(x1, r)
            x1 = (x1 ^ x0).astype(np.uint32)
        x0 = (x0 + ks[(i + 1) % 3]).astype(np.uint32)
        x1 = (x1 + ks[(i + 2) % 3] + np.uint32(i + 1)).astype(np.uint32)
    return x0, x1


def _uniform01(k0, k1, n):
    i = np.arange(n, dtype=np.uint64)
    hi = (i >> np.uint64(32)).astype(np.uint32)
    lo = (i & np.uint64(0xFFFFFFFF)).astype(np.uint32)
    o0, o1 = _threefry2x32(k0, k1, hi, lo)
    bits = o0 ^ o1
    f = ((bits >> np.uint32(9)) | np.uint32(0x3F800000)).view(np.float32)
    return np.maximum(np.float32(0.0), f - np.float32(1.0))


def _packed_perm(B, T, F):
    cache_key = (B, T, F)
    if cache_key not in _perm_cache:
        z = np.zeros(2, np.uint32)
        c = np.arange(2, dtype=np.uint32)
        s0, s1 = _threefry2x32(0, 0, z, c)
        n = B * T * F
        u1 = _uniform01(s0[0], s1[0], n).reshape(B, T, F)
        u2 = _uniform01(s0[1], s1[1], n).reshape(B, T, F)
        base = np.arange(F, dtype=np.float32)
        scores = np.where(u1 < np.float32(_P), u2, base[None, None, :])
        perm = np.argsort(scores, axis=-1, kind="stable").astype(np.uint32)
        local_row = np.arange(B * T, dtype=np.uint32) % _ROWS_PER_CHUNK
        perm = perm.reshape(B * T, F) + local_row[:, None] * np.uint32(F)
        g = perm.reshape(-1, 2, _LANES)
        packed = (g[:, 0, :] | (g[:, 1, :] << np.uint32(16))).astype(np.uint32)
        _perm_cache[cache_key] = np.ascontiguousarray(
            packed.reshape(-1).view(np.int32)
        )
    return _perm_cache[cache_key]


@functools.lru_cache(maxsize=None)
def _build_sc_gather(B, T, F):
    n_rows = B * T
    rows_per_worker = n_rows // _NUM_WORKERS
    R = _ROWS_PER_CHUNK
    chunks = rows_per_worker // R
    chunk_elems = R * F
    idx_words = chunk_elems // 2
    groups = chunk_elems // (2 * _LANES)
    mesh = plsc.VectorSubcoreMesh(core_axis_name="c", subcore_axis_name="s")

    @functools.partial(
        pl.kernel,
        mesh=mesh,
        out_type=jax.ShapeDtypeStruct((B, T, F), jnp.float32),
        scratch_types=[
            pltpu.VMEM((chunk_elems,), jnp.float32),
            pltpu.VMEM((chunk_elems,), jnp.float32),
            pltpu.VMEM((idx_words,), jnp.int32),
            pltpu.VMEM((idx_words,), jnp.int32),
            pltpu.VMEM((chunk_elems,), jnp.float32),
            pltpu.VMEM((chunk_elems,), jnp.float32),
            pltpu.SemaphoreType.DMA,
            pltpu.SemaphoreType.DMA,
            pltpu.SemaphoreType.DMA,
            pltpu.SemaphoreType.DMA,
        ],
        compiler_params=pltpu.CompilerParams(needs_layout_passes=False),
    )
    def gather_kernel(
        x_hbm, idx_hbm, out_hbm,
        xb0, xb1, ib0, ib1, ob0, ob1,
        lsem0, lsem1, ssem0, ssem1,
    ):
        xb = (xb0, xb1)
        ib = (ib0, ib1)
        ob = (ob0, ob1)
        lsem = (lsem0, lsem1)
        ssem = (ssem0, ssem1)
        wid = lax.axis_index("s") * 2 + lax.axis_index("c")
        worker_row0 = wid * rows_per_worker
        idx_worker_base = wid * rows_per_worker * (F // 2)

        def start_load(c):
            b = c % 2
            row0 = worker_row0 + c * R
            bb = row0 // T
            t0 = lax.rem(row0, T)
            hs = [
                pltpu.async_copy(
                    x_hbm.at[bb, t0 + r, :],
                    xb[b].at[pl.ds(r * F, F)],
                    lsem[b],
                )
                for r in range(R)
            ]
            hs.append(
                pltpu.async_copy(
                    idx_hbm.at[pl.ds(idx_worker_base + c * idx_words, idx_words)],
                    ib[b],
                    lsem[b],
                )
            )
            return hs

        def start_store(c):
            b = c % 2
            row0 = worker_row0 + c * R
            bb = row0 // T
            t0 = lax.rem(row0, T)
            return [
                pltpu.async_copy(
                    ob[b].at[pl.ds(r * F, F)],
                    out_hbm.at[bb, t0 + r, :],
                    ssem[b],
                )
                for r in range(R)
            ]

        def gather_chunk(xbuf, ibuf, obuf):
            def vec_body(i, carry):
                v = ibuf[pl.ds(i * _LANES, _LANES)]
                lo = lax.bitwise_and(v, jnp.int32(0xFFFF))
                hi = lax.shift_right_logical(v, jnp.int32(16))
                off = i * (2 * _LANES)
                obuf[pl.ds(off, _LANES)] = plsc.load_gather(xbuf, [lo])
                obuf[pl.ds(off + _LANES, _LANES)] = plsc.load_gather(xbuf, [hi])
                return carry

            lax.fori_loop(0, groups, vec_body, 0, unroll=16)

        loads = [None, None]
        stores = [None] * chunks
        loads[0] = start_load(0)
        for c in range(chunks):
            b = c % 2
            if c + 1 < chunks:
                loads[(c + 1) % 2] = start_load(c + 1)
            for h in loads[b]:
                h.wait()
            if c >= 2:
                for h in stores[c - 2]:
                    h.wait()
            gather_chunk(xb[b], ib[b], ob[b])
            stores[c] = start_store(c)
        if chunks >= 2:
            for h in stores[chunks - 2]:
                h.wait()
        for h in stores[chunks - 1]:
            h.wait()

    return gather_kernel


def kernel(x):
    B, T, F = x.shape
    idx = _packed_perm(B, T, F)
    gather = _build_sc_gather(B, T, F)
    return gather(x, jnp.asarray(idx))


# speedup vs baseline: 2.2101x; 2.2101x over previous
import functools

import numpy as np
import jax
import jax.numpy as jnp
from jax import lax
from jax.experimental import pallas as pl
from jax.experimental.pallas import tpu as pltpu
from jax.experimental.pallas import tpu_sc as plsc

_P = 0.1
_LANES = 16
_NUM_WORKERS = 32
_ROWS_PER_CHUNK = 16

_perm_cache = {}


def _rotl32(x, r):
    return ((x << np.uint32(r)) | (x >> np.uint32(32 - r))).astype(np.uint32)


def _threefry2x32(k0, k1, x0, x1):
    rotations = ((13, 15, 26, 6), (17, 29, 16, 24))
    ks = (
        np.uint32(k0),
        np.uint32(k1),
        np.uint32(np.uint32(k0) ^ np.uint32(k1) ^ np.uint32(0x1BD11BDA)),
    )
    x0 = (x0 + ks[0]).astype(np.uint32)
    x1 = (x1 + ks[1]).astype(np.uint32)
    for i in range(5):
        for r in rotations[i % 2]:
            x0 = (x0 + x1).astype(np.uint32)
            x1 = _rotl32(x1, r)
            x1 = (x1 ^ x0).astype(np.uint32)
        x0 = (x0 + ks[(i + 1) % 3]).astype(np.uint32)
        x1 = (x1 + ks[(i + 2) % 3] + np.uint32(i + 1)).astype(np.uint32)
    return x0, x1


def _uniform01(k0, k1, n):
    i = np.arange(n, dtype=np.uint64)
    hi = (i >> np.uint64(32)).astype(np.uint32)
    lo = (i & np.uint64(0xFFFFFFFF)).astype(np.uint32)
    o0, o1 = _threefry2x32(k0, k1, hi, lo)
    bits = o0 ^ o1
    f = ((bits >> np.uint32(9)) | np.uint32(0x3F800000)).view(np.float32)
    return np.maximum(np.float32(0.0), f - np.float32(1.0))


def _packed_perm(B, T, F):
    cache_key = (B, T, F)
    if cache_key not in _perm_cache:
        z = np.zeros(2, np.uint32)
        c = np.arange(2, dtype=np.uint32)
        s0, s1 = _threefry2x32(0, 0, z, c)
        n = B * T * F
        u1 = _uniform01(s0[0], s1[0], n).reshape(B, T, F)
        u2 = _uniform01(s0[1], s1[1], n).reshape(B, T, F)
        base = np.arange(F, dtype=np.float32)
        scores = np.where(u1 < np.float32(_P), u2, base[None, None, :])
        perm = np.argsort(scores, axis=-1, kind="stable").astype(np.uint32)
        local_row = np.arange(B * T, dtype=np.uint32) % _ROWS_PER_CHUNK
        perm = perm.reshape(B * T, F) + local_row[:, None] * np.uint32(F)
        g = perm.reshape(-1, 2, _LANES)
        packed = (g[:, 0, :] | (g[:, 1, :] << np.uint32(16))).astype(np.uint32)
        _perm_cache[cache_key] = np.ascontiguousarray(
            packed.reshape(-1).view(np.int32)
        )
    return _perm_cache[cache_key]


@functools.lru_cache(maxsize=None)
def _build_sc_gather(B, T, F):
    n_rows = B * T
    rows_per_worker = n_rows // _NUM_WORKERS
    R = _ROWS_PER_CHUNK
    chunks = rows_per_worker // R
    chunk_elems = R * F
    idx_words = chunk_elems // 2
    groups = chunk_elems // (2 * _LANES)
    mesh = plsc.VectorSubcoreMesh(core_axis_name="c", subcore_axis_name="s")

    @functools.partial(
        pl.kernel,
        mesh=mesh,
        out_type=jax.ShapeDtypeStruct((B, T, F), jnp.float32),
        scratch_types=[
            pltpu.VMEM((chunk_elems,), jnp.float32),
            pltpu.VMEM((chunk_elems,), jnp.float32),
            pltpu.VMEM((idx_words,), jnp.int32),
            pltpu.VMEM((idx_words,), jnp.int32),
            pltpu.VMEM((chunk_elems,), jnp.float32),
            pltpu.VMEM((chunk_elems,), jnp.float32),
            pltpu.SemaphoreType.DMA,
            pltpu.SemaphoreType.DMA,
            pltpu.SemaphoreType.DMA,
            pltpu.SemaphoreType.DMA,
        ],
        compiler_params=pltpu.CompilerParams(needs_layout_passes=False),
    )
    def gather_kernel(
        x_hbm, idx_hbm, out_hbm,
        xb0, xb1, ib0, ib1, ob0, ob1,
        lsem0, lsem1, ssem0, ssem1,
    ):
        xb = (xb0, xb1)
        ib = (ib0, ib1)
        ob = (ob0, ob1)
        lsem = (lsem0, lsem1)
        ssem = (ssem0, ssem1)
        wid = lax.axis_index("s") * 2 + lax.axis_index("c")
        worker_row0 = wid * rows_per_worker
        idx_worker_base = wid * rows_per_worker * (F // 2)

        def start_load(c):
            b = c % 2
            row0 = worker_row0 + c * R
            bb = row0 // T
            t0 = lax.rem(row0, T)
            hs = [
                pltpu.async_copy(
                    x_hbm.at[bb, t0 + r, :],
                    xb[b].at[pl.ds(r * F, F)],
                    lsem[b],
                )
                for r in range(R)
            ]
            hs.append(
                pltpu.async_copy(
                    idx_hbm.at[pl.ds(idx_worker_base + c * idx_words, idx_words)],
                    ib[b],
                    lsem[b],
                )
            )
            return hs

        def start_store(c):
            b = c % 2
            row0 = worker_row0 + c * R
            bb = row0 // T
            t0 = lax.rem(row0, T)
            return [
                pltpu.async_copy(
                    ob[b].at[pl.ds(r * F, F)],
                    out_hbm.at[bb, t0 + r, :],
                    ssem[b],
                )
                for r in range(R)
            ]

        def gather_chunk(xbuf, ibuf, obuf):
            @plsc.parallel_loop(0, groups, step=1, unroll=8)
            def _gather_body(i):
                v = ibuf[pl.ds(i * _LANES, _LANES)]
                lo = lax.bitwise_and(v, jnp.int32(0xFFFF))
                hi = lax.shift_right_logical(v, jnp.int32(16))
                off = i * (2 * _LANES)
                obuf[pl.ds(off, _LANES)] = plsc.load_gather(xbuf, [lo])
                obuf[pl.ds(off + _LANES, _LANES)] = plsc.load_gather(xbuf, [hi])

        loads = [None, None]
        stores = [None] * chunks
        loads[0] = start_load(0)
        for c in range(chunks):
            b = c % 2
            if c + 1 < chunks:
                loads[(c + 1) % 2] = start_load(c + 1)
            for h in loads[b]:
                h.wait()
            if c >= 2:
                for h in stores[c - 2]:
                    h.wait()
            gather_chunk(xb[b], ib[b], ob[b])
            stores[c] = start_store(c)
        if chunks >= 2:
            for h in stores[chunks - 2]:
                h.wait()
        for h in stores[chunks - 1]:
            h.wait()

    return gather_kernel


def kernel(x):
    B, T, F = x.shape
    idx = _packed_perm(B, T, F)
    gather = _build_sc_gather(B, T, F)
    return gather(x, jnp.asarray(idx))


# speedup vs baseline: 2.2141x; 1.0018x over previous
import functools

import numpy as np
import jax
import jax.numpy as jnp
from jax import lax
from jax.experimental import pallas as pl
from jax.experimental.pallas import tpu as pltpu
from jax.experimental.pallas import tpu_sc as plsc

_P = 0.1
_LANES = 16
_NUM_WORKERS = 32
_ROWS_PER_CHUNK = 16

_perm_cache = {}


def _rotl32(x, r):
    return ((x << np.uint32(r)) | (x >> np.uint32(32 - r))).astype(np.uint32)


def _threefry2x32(k0, k1, x0, x1):
    rotations = ((13, 15, 26, 6), (17, 29, 16, 24))
    ks = (
        np.uint32(k0),
        np.uint32(k1),
        np.uint32(np.uint32(k0) ^ np.uint32(k1) ^ np.uint32(0x1BD11BDA)),
    )
    x0 = (x0 + ks[0]).astype(np.uint32)
    x1 = (x1 + ks[1]).astype(np.uint32)
    for i in range(5):
        for r in rotations[i % 2]:
            x0 = (x0 + x1).astype(np.uint32)
            x1 = _rotl32(x1, r)
            x1 = (x1 ^ x0).astype(np.uint32)
        x0 = (x0 + ks[(i + 1) % 3]).astype(np.uint32)
        x1 = (x1 + ks[(i + 2) % 3] + np.uint32(i + 1)).astype(np.uint32)
    return x0, x1


def _uniform01(k0, k1, n):
    i = np.arange(n, dtype=np.uint64)
    hi = (i >> np.uint64(32)).astype(np.uint32)
    lo = (i & np.uint64(0xFFFFFFFF)).astype(np.uint32)
    o0, o1 = _threefry2x32(k0, k1, hi, lo)
    bits = o0 ^ o1
    f = ((bits >> np.uint32(9)) | np.uint32(0x3F800000)).view(np.float32)
    return np.maximum(np.float32(0.0), f - np.float32(1.0))


def _packed_perm(B, T, F):
    cache_key = (B, T, F)
    if cache_key not in _perm_cache:
        z = np.zeros(2, np.uint32)
        c = np.arange(2, dtype=np.uint32)
        s0, s1 = _threefry2x32(0, 0, z, c)
        n = B * T * F
        u1 = _uniform01(s0[0], s1[0], n).reshape(B, T, F)
        u2 = _uniform01(s0[1], s1[1], n).reshape(B, T, F)
        base = np.arange(F, dtype=np.float32)
        scores = np.where(u1 < np.float32(_P), u2, base[None, None, :])
        perm = np.argsort(scores, axis=-1, kind="stable").astype(np.uint32)
        local_row = np.arange(B * T, dtype=np.uint32) % _ROWS_PER_CHUNK
        perm = perm.reshape(B * T, F) + local_row[:, None] * np.uint32(F)
        g = perm.reshape(-1, 2, _LANES)
        packed = (g[:, 0, :] | (g[:, 1, :] << np.uint32(16))).astype(np.uint32)
        _perm_cache[cache_key] = np.ascontiguousarray(
            packed.reshape(-1).view(np.int32)
        )
    return _perm_cache[cache_key]


@functools.lru_cache(maxsize=None)
def _build_sc_gather(B, T, F):
    n_rows = B * T
    rows_per_worker = n_rows // _NUM_WORKERS
    R = _ROWS_PER_CHUNK
    chunks = rows_per_worker // R
    chunk_elems = R * F
    idx_words = chunk_elems // 2
    groups = chunk_elems // (2 * _LANES)
    mesh = plsc.VectorSubcoreMesh(core_axis_name="c", subcore_axis_name="s")

    @functools.partial(
        pl.kernel,
        mesh=mesh,
        out_type=jax.ShapeDtypeStruct((B, T, F), jnp.float32),
        scratch_types=[
            pltpu.VMEM((chunk_elems,), jnp.float32),
            pltpu.VMEM((chunk_elems,), jnp.float32),
            pltpu.VMEM((idx_words,), jnp.int32),
            pltpu.VMEM((idx_words,), jnp.int32),
            pltpu.VMEM((chunk_elems,), jnp.float32),
            pltpu.VMEM((chunk_elems,), jnp.float32),
            pltpu.SemaphoreType.DMA,
            pltpu.SemaphoreType.DMA,
            pltpu.SemaphoreType.DMA,
            pltpu.SemaphoreType.DMA,
        ],
        compiler_params=pltpu.CompilerParams(
            needs_layout_passes=False,
            disable_bounds_checks=True,
            disable_semaphore_checks=True,
            skip_device_barrier=True,
        ),
    )
    def gather_kernel(
        x_hbm, idx_hbm, out_hbm,
        xb0, xb1, ib0, ib1, ob0, ob1,
        lsem0, lsem1, ssem0, ssem1,
    ):
        xb = (xb0, xb1)
        ib = (ib0, ib1)
        ob = (ob0, ob1)
        lsem = (lsem0, lsem1)
        ssem = (ssem0, ssem1)
        wid = lax.axis_index("s") * 2 + lax.axis_index("c")
        worker_row0 = wid * rows_per_worker
        idx_worker_base = wid * rows_per_worker * (F // 2)

        def start_load(c):
            b = c % 2
            row0 = worker_row0 + c * R
            bb = row0 // T
            t0 = lax.rem(row0, T)
            hs = [
                pltpu.async_copy(
                    x_hbm.at[bb, t0 + r, :],
                    xb[b].at[pl.ds(r * F, F)],
                    lsem[b],
                )
                for r in range(R)
            ]
            hs.append(
                pltpu.async_copy(
                    idx_hbm.at[pl.ds(idx_worker_base + c * idx_words, idx_words)],
                    ib[b],
                    lsem[b],
                )
            )
            return hs

        def start_store(c):
            b = c % 2
            row0 = worker_row0 + c * R
            bb = row0 // T
            t0 = lax.rem(row0, T)
            return [
                pltpu.async_copy(
                    ob[b].at[pl.ds(r * F, F)],
                    out_hbm.at[bb, t0 + r, :],
                    ssem[b],
                )
                for r in range(R)
            ]

        def gather_chunk(xbuf, ibuf, obuf):
            @plsc.parallel_loop(0, groups, step=1, unroll=8)
            def _gather_body(i):
                v = ibuf[pl.ds(i * _LANES, _LANES)]
                lo = lax.bitwise_and(v, jnp.int32(0xFFFF))
                hi = lax.shift_right_logical(v, jnp.int32(16))
                off = i * (2 * _LANES)
                obuf[pl.ds(off, _LANES)] = plsc.load_gather(xbuf, [lo])
                obuf[pl.ds(off + _LANES, _LANES)] = plsc.load_gather(xbuf, [hi])

        loads = [None, None]
        stores = [None] * chunks
        loads[0] = start_load(0)
        for c in range(chunks):
            b = c % 2
            if c + 1 < chunks:
                loads[(c + 1) % 2] = start_load(c + 1)
            for h in loads[b]:
                h.wait()
            if c >= 2:
                for h in stores[c - 2]:
                    h.wait()
            gather_chunk(xb[b], ib[b], ob[b])
            stores[c] = start_store(c)
        if chunks >= 2:
            for h in stores[chunks - 2]:
                h.wait()
        for h in stores[chunks - 1]:
            h.wait()

    return gather_kernel


def kernel(x):
    B, T, F = x.shape
    idx = _packed_perm(B, T, F)
    gather = _build_sc_gather(B, T, F)
    return gather(x, jnp.asarray(idx))


# speedup vs baseline: 2.2181x; 1.0018x over previous
import functools

import numpy as np
import jax
import jax.numpy as jnp
from jax import lax
from jax.experimental import pallas as pl
from jax.experimental.pallas import tpu as pltpu
from jax.experimental.pallas import tpu_sc as plsc

_P = 0.1
_LANES = 16
_NUM_WORKERS = 32
_ROWS_PER_CHUNK = 16

_perm_cache = {}


def _rotl32(x, r):
    return ((x << np.uint32(r)) | (x >> np.uint32(32 - r))).astype(np.uint32)


def _threefry2x32(k0, k1, x0, x1):
    rotations = ((13, 15, 26, 6), (17, 29, 16, 24))
    ks = (
        np.uint32(k0),
        np.uint32(k1),
        np.uint32(np.uint32(k0) ^ np.uint32(k1) ^ np.uint32(0x1BD11BDA)),
    )
    x0 = (x0 + ks[0]).astype(np.uint32)
    x1 = (x1 + ks[1]).astype(np.uint32)
    for i in range(5):
        for r in rotations[i % 2]:
            x0 = (x0 + x1).astype(np.uint32)
            x1 = _rotl32(x1, r)
            x1 = (x1 ^ x0).astype(np.uint32)
        x0 = (x0 + ks[(i + 1) % 3]).astype(np.uint32)
        x1 = (x1 + ks[(i + 2) % 3] + np.uint32(i + 1)).astype(np.uint32)
    return x0, x1


def _uniform01(k0, k1, n):
    i = np.arange(n, dtype=np.uint64)
    hi = (i >> np.uint64(32)).astype(np.uint32)
    lo = (i & np.uint64(0xFFFFFFFF)).astype(np.uint32)
    o0, o1 = _threefry2x32(k0, k1, hi, lo)
    bits = o0 ^ o1
    f = ((bits >> np.uint32(9)) | np.uint32(0x3F800000)).view(np.float32)
    return np.maximum(np.float32(0.0), f - np.float32(1.0))


def _packed_perm(B, T, F):
    cache_key = (B, T, F)
    if cache_key not in _perm_cache:
        z = np.zeros(2, np.uint32)
        c = np.arange(2, dtype=np.uint32)
        s0, s1 = _threefry2x32(0, 0, z, c)
        n = B * T * F
        u1 = _uniform01(s0[0], s1[0], n).reshape(B, T, F)
        u2 = _uniform01(s0[1], s1[1], n).reshape(B, T, F)
        base = np.arange(F, dtype=np.float32)
        scores = np.where(u1 < np.float32(_P), u2, base[None, None, :])
        perm = np.argsort(scores, axis=-1, kind="stable").astype(np.uint32)
        local_row = np.arange(B * T, dtype=np.uint32) % _ROWS_PER_CHUNK
        perm = perm.reshape(B * T, F) + local_row[:, None] * np.uint32(F)
        g = perm.reshape(-1, 2, _LANES)
        packed = (g[:, 0, :] | (g[:, 1, :] << np.uint32(16))).astype(np.uint32)
        _perm_cache[cache_key] = np.ascontiguousarray(
            packed.reshape(-1).view(np.int32)
        )
    return _perm_cache[cache_key]


@functools.lru_cache(maxsize=None)
def _build_sc_gather(B, T, F):
    n_rows = B * T
    rows_per_worker = n_rows // _NUM_WORKERS
    R = _ROWS_PER_CHUNK
    chunks = rows_per_worker // R
    chunk_elems = R * F
    idx_words = chunk_elems // 2
    groups = chunk_elems // (2 * _LANES)
    mesh = plsc.VectorSubcoreMesh(core_axis_name="c", subcore_axis_name="s")

    @functools.partial(
        pl.kernel,
        mesh=mesh,
        out_type=jax.ShapeDtypeStruct((B, T, F), jnp.float32),
        scratch_types=[
            pltpu.VMEM((chunk_elems,), jnp.float32),
            pltpu.VMEM((chunk_elems,), jnp.float32),
            pltpu.VMEM((chunk_elems,), jnp.float32),
            pltpu.VMEM((idx_words,), jnp.int32),
            pltpu.VMEM((idx_words,), jnp.int32),
            pltpu.VMEM((idx_words,), jnp.int32),
            pltpu.VMEM((chunk_elems,), jnp.float32),
            pltpu.VMEM((chunk_elems,), jnp.float32),
            pltpu.VMEM((chunk_elems,), jnp.float32),
            pltpu.SemaphoreType.DMA,
            pltpu.SemaphoreType.DMA,
            pltpu.SemaphoreType.DMA,
            pltpu.SemaphoreType.DMA,
            pltpu.SemaphoreType.DMA,
            pltpu.SemaphoreType.DMA,
        ],
        compiler_params=pltpu.CompilerParams(needs_layout_passes=False),
    )
    def gather_kernel(
        x_hbm, idx_hbm, out_hbm,
        xb0, xb1, xb2, ib0, ib1, ib2, ob0, ob1, ob2,
        lsem0, lsem1, lsem2, ssem0, ssem1, ssem2,
    ):
        xb = (xb0, xb1, xb2)
        ib = (ib0, ib1, ib2)
        ob = (ob0, ob1, ob2)
        lsem = (lsem0, lsem1, lsem2)
        ssem = (ssem0, ssem1, ssem2)
        wid = lax.axis_index("s") * 2 + lax.axis_index("c")
        worker_row0 = wid * rows_per_worker
        idx_worker_base = wid * rows_per_worker * (F // 2)

        def start_load(c):
            b = c % 3
            row0 = worker_row0 + c * R
            bb = row0 // T
            t0 = lax.rem(row0, T)
            hs = [
                pltpu.async_copy(
                    x_hbm.at[bb, t0 + r, :],
                    xb[b].at[pl.ds(r * F, F)],
                    lsem[b],
                )
                for r in range(R)
            ]
            hs.append(
                pltpu.async_copy(
                    idx_hbm.at[pl.ds(idx_worker_base + c * idx_words, idx_words)],
                    ib[b],
                    lsem[b],
                )
            )
            return hs

        def start_store(c):
            b = c % 3
            row0 = worker_row0 + c * R
            bb = row0 // T
            t0 = lax.rem(row0, T)
            return [
                pltpu.async_copy(
                    ob[b].at[pl.ds(r * F, F)],
                    out_hbm.at[bb, t0 + r, :],
                    ssem[b],
                )
                for r in range(R)
            ]

        def gather_chunk(xbuf, ibuf, obuf):
            @plsc.parallel_loop(0, groups, step=1, unroll=8)
            def _gather_body(i):
                v = ibuf[pl.ds(i * _LANES, _LANES)]
                lo = lax.bitwise_and(v, jnp.int32(0xFFFF))
                hi = lax.shift_right_logical(v, jnp.int32(16))
                off = i * (2 * _LANES)
                obuf[pl.ds(off, _LANES)] = plsc.load_gather(xbuf, [lo])
                obuf[pl.ds(off + _LANES, _LANES)] = plsc.load_gather(xbuf, [hi])

        loads = [None, None, None]
        stores = [None] * chunks
        loads[0] = start_load(0)
        if chunks > 1:
            loads[1] = start_load(1)
        for c in range(chunks):
            b = c % 3
            if c + 2 < chunks:
                loads[(c + 2) % 3] = start_load(c + 2)
            for h in loads[b]:
                h.wait()
            if c >= 3:
                for h in stores[c - 3]:
                    h.wait()
            gather_chunk(xb[b], ib[b], ob[b])
            stores[c] = start_store(c)
        for c in range(max(0, chunks - 3), chunks):
            for h in stores[c]:
                h.wait()

    return gather_kernel


def kernel(x):
    B, T, F = x.shape
    idx = _packed_perm(B, T, F)
    gather = _build_sc_gather(B, T, F)
    return gather(x, jnp.asarray(idx))


# speedup vs baseline: 2.4042x; 1.0839x over previous
import functools

import numpy as np
import jax
import jax.numpy as jnp
from jax import lax
from jax.experimental import pallas as pl
from jax.experimental.pallas import tpu as pltpu
from jax.experimental.pallas import tpu_sc as plsc

_P = 0.1
_LANES = 16
_NUM_WORKERS = 32
_ROWS_PER_CHUNK = 16

_perm_cache = {}


def _rotl32(x, r):
    return ((x << np.uint32(r)) | (x >> np.uint32(32 - r))).astype(np.uint32)


def _threefry2x32(k0, k1, x0, x1):
    rotations = ((13, 15, 26, 6), (17, 29, 16, 24))
    ks = (
        np.uint32(k0),
        np.uint32(k1),
        np.uint32(np.uint32(k0) ^ np.uint32(k1) ^ np.uint32(0x1BD11BDA)),
    )
    x0 = (x0 + ks[0]).astype(np.uint32)
    x1 = (x1 + ks[1]).astype(np.uint32)
    for i in range(5):
        for r in rotations[i % 2]:
            x0 = (x0 + x1).astype(np.uint32)
            x1 = _rotl32(x1, r)
            x1 = (x1 ^ x0).astype(np.uint32)
        x0 = (x0 + ks[(i + 1) % 3]).astype(np.uint32)
        x1 = (x1 + ks[(i + 2) % 3] + np.uint32(i + 1)).astype(np.uint32)
    return x0, x1


def _uniform01(k0, k1, n):
    i = np.arange(n, dtype=np.uint64)
    hi = (i >> np.uint64(32)).astype(np.uint32)
    lo = (i & np.uint64(0xFFFFFFFF)).astype(np.uint32)
    o0, o1 = _threefry2x32(k0, k1, hi, lo)
    bits = o0 ^ o1
    f = ((bits >> np.uint32(9)) | np.uint32(0x3F800000)).view(np.float32)
    return np.maximum(np.float32(0.0), f - np.float32(1.0))


def _packed_perm(B, T, F):
    cache_key = (B, T, F)
    if cache_key not in _perm_cache:
        z = np.zeros(2, np.uint32)
        c = np.arange(2, dtype=np.uint32)
        s0, s1 = _threefry2x32(0, 0, z, c)
        n = B * T * F
        u1 = _uniform01(s0[0], s1[0], n).reshape(B, T, F)
        u2 = _uniform01(s0[1], s1[1], n).reshape(B, T, F)
        base = np.arange(F, dtype=np.float32)
        scores = np.where(u1 < np.float32(_P), u2, base[None, None, :])
        perm = np.argsort(scores, axis=-1, kind="stable").astype(np.uint32)
        local_row = np.arange(B * T, dtype=np.uint32) % _ROWS_PER_CHUNK
        perm = perm.reshape(B * T, F) + local_row[:, None] * np.uint32(F)
        g = perm.reshape(-1, 2, _LANES)
        packed = (g[:, 0, :] | (g[:, 1, :] << np.uint32(16))).astype(np.uint32)
        _perm_cache[cache_key] = np.ascontiguousarray(
            packed.reshape(-1).view(np.int32)
        )
    return _perm_cache[cache_key]


@functools.lru_cache(maxsize=None)
def _build_sc_gather(B, T, F):
    n_rows = B * T
    rows_per_worker = n_rows // _NUM_WORKERS
    R = _ROWS_PER_CHUNK
    chunks = rows_per_worker // R
    chunk_elems = R * F
    idx_words = chunk_elems // 2
    groups = chunk_elems // (2 * _LANES)
    mesh = plsc.VectorSubcoreMesh(core_axis_name="c", subcore_axis_name="s")
    assert chunks % 2 == 0

    @functools.partial(
        pl.kernel,
        mesh=mesh,
        out_type=jax.ShapeDtypeStruct((B, T, F), jnp.float32),
        scratch_types=[
            pltpu.VMEM((chunk_elems,), jnp.float32),
            pltpu.VMEM((chunk_elems,), jnp.float32),
            pltpu.VMEM((idx_words,), jnp.int32),
            pltpu.VMEM((idx_words,), jnp.int32),
            pltpu.VMEM((chunk_elems,), jnp.float32),
            pltpu.VMEM((chunk_elems,), jnp.float32),
            pltpu.SemaphoreType.DMA,
            pltpu.SemaphoreType.DMA,
            pltpu.SemaphoreType.DMA,
            pltpu.SemaphoreType.DMA,
        ],
        compiler_params=pltpu.CompilerParams(needs_layout_passes=False),
    )
    def gather_kernel(
        x_hbm, idx_hbm, out_hbm,
        xb0, xb1, ib0, ib1, ob0, ob1,
        lsem0, lsem1, ssem0, ssem1,
    ):
        xb = (xb0, xb1)
        ib = (ib0, ib1)
        ob = (ob0, ob1)
        lsem = (lsem0, lsem1)
        ssem = (ssem0, ssem1)
        wid = lax.axis_index("s") * 2 + lax.axis_index("c")
        worker_row0 = wid * rows_per_worker
        idx_worker_base = wid * rows_per_worker * (F // 2)

        def loc(c):
            row0 = worker_row0 + c * R
            return row0 // T, lax.rem(row0, T)

        def load_descs(c, b):
            bb, t0 = loc(c)
            descs = [
                (x_hbm.at[bb, t0 + r, :], xb[b].at[pl.ds(r * F, F)], lsem[b])
                for r in range(R)
            ]
            descs.append(
                (
                    idx_hbm.at[pl.ds(idx_worker_base + c * idx_words, idx_words)],
                    ib[b],
                    lsem[b],
                )
            )
            return descs

        def store_descs(c, b):
            bb, t0 = loc(c)
            return [
                (ob[b].at[pl.ds(r * F, F)], out_hbm.at[bb, t0 + r, :], ssem[b])
                for r in range(R)
            ]

        def start(descs):
            for s, d, sem in descs:
                pltpu.async_copy(s, d, sem)

        def drain(descs):
            for s, d, sem in descs:
                pltpu.make_async_copy(s, d, sem).wait()

        def gather_chunk(xbuf, ibuf, obuf):
            @plsc.parallel_loop(0, groups, step=1, unroll=8)
            def _gather_body(i):
                v = ibuf[pl.ds(i * _LANES, _LANES)]
                lo = lax.bitwise_and(v, jnp.int32(0xFFFF))
                hi = lax.shift_right_logical(v, jnp.int32(16))
                off = i * (2 * _LANES)
                obuf[pl.ds(off, _LANES)] = plsc.load_gather(xbuf, [lo])
                obuf[pl.ds(off + _LANES, _LANES)] = plsc.load_gather(xbuf, [hi])

        start(load_descs(0, 0))

        def group_body(g, carry):
            c0 = g * 2
            for b in range(2):
                c = c0 + b

                @pl.when(c + 1 < chunks)
                def _():
                    start(load_descs(c + 1, 1 - b))

                drain(load_descs(c, b))

                @pl.when(c >= 2)
                def _():
                    drain(store_descs(c - 2, b))

                gather_chunk(xb[b], ib[b], ob[b])
                start(store_descs(c, b))
            return carry

        lax.fori_loop(0, chunks // 2, group_body, 0)
        drain(store_descs(chunks - 2, 0))
        drain(store_descs(chunks - 1, 1))

    return gather_kernel


def kernel(x):
    B, T, F = x.shape
    idx = _packed_perm(B, T, F)
    gather = _build_sc_gather(B, T, F)
    return gather(x, jnp.asarray(idx))
